# baseline (speedup 1.0000x reference)
import jax
import jax.numpy as jnp
from jax import lax
from jax.experimental import pallas as pl
from jax.experimental.pallas import tpu as pltpu

HIDDEN = 768
NUM_EXPERTS = 8
TOP_K = 2
D_FF = HIDDEN * 2
BM = 256
CH = 512


def _ffn_kernel(te_ref, tv_ref, xs_ref, w1_ref, b1_ref, w2_ref, b2_ref,
                out_ref):
    i = pl.program_id(0)

    @pl.when(tv_ref[i] > 0)
    def _():
        x_t = xs_ref[...].astype(jnp.bfloat16)
        h = jnp.dot(x_t, w1_ref[0].astype(jnp.bfloat16),
                    preferred_element_type=jnp.float32)
        h = jnp.maximum(h + b1_ref[0, 0, :][None, :], 0.0)
        o = jnp.dot(h.astype(jnp.bfloat16), w2_ref[0].astype(jnp.bfloat16),
                    preferred_element_type=jnp.float32)
        out_ref[...] = o + b2_ref[0, 0, :][None, :]


def _grouped_ffn(xs, w1, b1, w2, b2, tile_expert, tile_valid, ntiles):
    grid_spec = pltpu.PrefetchScalarGridSpec(
        num_scalar_prefetch=2,
        grid=(ntiles,),
        in_specs=[
            pl.BlockSpec((BM, HIDDEN), lambda i, te, tv: (i, 0)),
            pl.BlockSpec((1, HIDDEN, D_FF), lambda i, te, tv: (te[i], 0, 0)),
            pl.BlockSpec((1, 1, D_FF), lambda i, te, tv: (te[i], 0, 0)),
            pl.BlockSpec((1, D_FF, HIDDEN), lambda i, te, tv: (te[i], 0, 0)),
            pl.BlockSpec((1, 1, HIDDEN), lambda i, te, tv: (te[i], 0, 0)),
        ],
        out_specs=pl.BlockSpec((BM, HIDDEN), lambda i, te, tv: (i, 0)),
    )
    return pl.pallas_call(
        _ffn_kernel,
        grid_spec=grid_spec,
        out_shape=jax.ShapeDtypeStruct((ntiles * BM, HIDDEN), jnp.float32),
    )(tile_expert, tile_valid, xs, w1, b1[:, None, :], w2, b2[:, None, :])


@jax.jit
def kernel(x, router_w, router_b, w1, b1, w2, b2):
    B, S, H = x.shape
    T = B * S
    A = T * TOP_K
    P = A + NUM_EXPERTS * BM
    ntiles = P // BM

    xf = x.reshape(T, H)

    logits = xf @ router_w + router_b
    probs = jax.nn.softmax(logits, axis=-1)
    top_p, top_i = lax.top_k(probs, TOP_K)
    top_p = top_p / jnp.sum(top_p, axis=-1, keepdims=True)

    expert_id = top_i.reshape(A)

    onehot = (expert_id[:, None] == jnp.arange(NUM_EXPERTS)[None, :])
    ohb = onehot.astype(jnp.bfloat16).reshape(A // CH, CH, NUM_EXPERTS)
    tri = jnp.tril(jnp.ones((CH, CH), jnp.bfloat16), -1)
    partial = jnp.einsum('st,cte->cse', tri, ohb,
                         preferred_element_type=jnp.float32)
    chunk_tot = jnp.sum(ohb, axis=1, dtype=jnp.float32)
    prefix = jnp.concatenate(
        [jnp.zeros((1, NUM_EXPERTS), jnp.float32),
         jnp.cumsum(chunk_tot, axis=0)[:-1]], axis=0)
    rank_full = (partial + prefix[:, None, :]).reshape(A, NUM_EXPERTS)
    rank = jnp.sum(rank_full * onehot, axis=1).astype(jnp.int32)
    counts = jnp.sum(chunk_tot, axis=0).astype(jnp.int32)

    padded_counts = ((counts + BM - 1) // BM) * BM
    padded_offsets = jnp.concatenate(
        [jnp.zeros((1,), jnp.int32), jnp.cumsum(padded_counts)[:-1]])
    pos = padded_offsets[expert_id] + rank

    token_of_pos = jnp.zeros((P,), jnp.int32).at[pos].set(
        jnp.arange(A, dtype=jnp.int32) // TOP_K)

    starts = padded_offsets // BM
    ends = (padded_offsets + padded_counts) // BM
    tid = jnp.arange(ntiles, dtype=jnp.int32)
    in_e = (tid[:, None] >= starts[None, :]) & (tid[:, None] < ends[None, :])
    tile_valid = jnp.any(in_e, axis=1).astype(jnp.int32)
    tile_expert = jnp.sum(
        in_e.astype(jnp.int32) * jnp.arange(NUM_EXPERTS)[None, :],
        axis=1).astype(jnp.int32)

    xs = xf[token_of_pos]
    eo = _grouped_ffn(xs, w1, b1, w2, b2, tile_expert, tile_valid,
                      ntiles)

    pos2 = pos.reshape(T, TOP_K)
    out = jnp.sum(eo[pos2] * top_p[..., None], axis=1)
    return out.reshape(B, S, H)


# speedup vs baseline: 1.2605x; 1.2605x over previous
import jax
import jax.numpy as jnp
from jax import lax
from jax.experimental import pallas as pl
from jax.experimental.pallas import tpu as pltpu

HIDDEN = 768
NUM_EXPERTS = 8
TOP_K = 2
D_FF = HIDDEN * 2
BM = 256
CH = 512
T_TOK = 2048
A_TOT = T_TOK * TOP_K
P_PAD = A_TOT + NUM_EXPERTS * BM
NTILES = P_PAD // BM


def _meta_kernel(ti_ref, pos_ref, te_ref, tv_ref, rank_ref):
    T, E, e_i32 = T_TOK, NUM_EXPERTS, jnp.int32

    iota = lax.broadcasted_iota(e_i32, (T, E), 1)
    e1 = ti_ref[:, 0:1]
    e2 = ti_ref[:, 1:2]
    oh1 = (iota == e1).astype(jnp.float32)
    oh2 = (iota == e2).astype(jnp.float32)

    tri = (lax.broadcasted_iota(e_i32, (CH, CH), 0)
           > lax.broadcasted_iota(e_i32, (CH, CH), 1)).astype(jnp.bfloat16)
    base = jnp.zeros((1, E), jnp.float32)
    for c in range(A_TOT // CH):
        src = oh1 if c < T // CH else oh2
        ohf = src[(c % (T // CH)) * CH:((c % (T // CH)) + 1) * CH, :]
        partial = jnp.dot(tri, ohf.astype(jnp.bfloat16),
                          preferred_element_type=jnp.float32)
        rank_c = partial + base
        rank_ref[c * CH:(c + 1) * CH, :] = (
            jnp.sum(rank_c * ohf, axis=1, keepdims=True))
        base = base + jnp.sum(ohf, axis=0, keepdims=True)

    off = jnp.int32(0)
    starts, ends, offs = [], [], []
    for e in range(E):
        c_e = base[0, e].astype(e_i32)
        pc = ((c_e + BM - 1) // BM) * BM
        offs.append(off)
        starts.append(off // BM)
        ends.append((off + pc) // BM)
        off = off + pc
    for t in range(NTILES):
        te_t = jnp.int32(0)
        tv_t = jnp.int32(0)
        for e in range(E):
            inside = ((t >= starts[e]) & (t < ends[e])).astype(e_i32)
            te_t = te_t + inside * e
            tv_t = tv_t | inside
        te_ref[t] = te_t
        tv_ref[t] = tv_t

    iota8 = lax.broadcasted_iota(e_i32, (1, E), 1)
    offv = jnp.zeros((1, E), jnp.float32)
    for e in range(E):
        offv = jnp.where(iota8 == e, offs[e].astype(jnp.float32), offv)
    off1 = jnp.sum(oh1 * offv, axis=1, keepdims=True)
    off2 = jnp.sum(oh2 * offv, axis=1, keepdims=True)
    pos_ref[0:T, :] = (rank_ref[0:T, :] + off1).astype(e_i32)
    pos_ref[T:2 * T, :] = (rank_ref[T:2 * T, :] + off2).astype(e_i32)


def _meta(top_i):
    return pl.pallas_call(
        _meta_kernel,
        out_shape=[
            jax.ShapeDtypeStruct((A_TOT, 1), jnp.int32),
            jax.ShapeDtypeStruct((NTILES,), jnp.int32),
            jax.ShapeDtypeStruct((NTILES,), jnp.int32),
        ],
        out_specs=[
            pl.BlockSpec(memory_space=pltpu.VMEM),
            pl.BlockSpec(memory_space=pltpu.SMEM),
            pl.BlockSpec(memory_space=pltpu.SMEM),
        ],
        scratch_shapes=[pltpu.VMEM((A_TOT, 1), jnp.float32)],
    )(top_i)


def _ffn_kernel(te_ref, tv_ref, xs_ref, w1_ref, b1_ref, w2_ref, b2_ref,
                out_ref):
    i = pl.program_id(0)

    @pl.when(tv_ref[i] > 0)
    def _():
        x_t = xs_ref[...].astype(jnp.bfloat16)
        h = jnp.dot(x_t, w1_ref[0].astype(jnp.bfloat16),
                    preferred_element_type=jnp.float32)
        h = jnp.maximum(h + b1_ref[0, 0, :][None, :], 0.0)
        o = jnp.dot(h.astype(jnp.bfloat16), w2_ref[0].astype(jnp.bfloat16),
                    preferred_element_type=jnp.float32)
        out_ref[...] = o + b2_ref[0, 0, :][None, :]


def _grouped_ffn(xs, w1, b1, w2, b2, tile_expert, tile_valid):
    grid_spec = pltpu.PrefetchScalarGridSpec(
        num_scalar_prefetch=2,
        grid=(NTILES,),
        in_specs=[
            pl.BlockSpec((BM, HIDDEN), lambda i, te, tv: (i, 0)),
            pl.BlockSpec((1, HIDDEN, D_FF), lambda i, te, tv: (te[i], 0, 0)),
            pl.BlockSpec((1, 1, D_FF), lambda i, te, tv: (te[i], 0, 0)),
            pl.BlockSpec((1, D_FF, HIDDEN), lambda i, te, tv: (te[i], 0, 0)),
            pl.BlockSpec((1, 1, HIDDEN), lambda i, te, tv: (te[i], 0, 0)),
        ],
        out_specs=pl.BlockSpec((BM, HIDDEN), lambda i, te, tv: (i, 0)),
    )
    return pl.pallas_call(
        _ffn_kernel,
        grid_spec=grid_spec,
        out_shape=jax.ShapeDtypeStruct((P_PAD, HIDDEN), jnp.float32),
    )(tile_expert, tile_valid, xs, w1, b1[:, None, :], w2, b2[:, None, :])


@jax.jit
def kernel(x, router_w, router_b, w1, b1, w2, b2):
    B, S, H = x.shape
    T = B * S
    xf = x.reshape(T, H)

    logits = xf @ router_w + router_b
    probs = jax.nn.softmax(logits, axis=-1)
    top_p, top_i = lax.top_k(probs, TOP_K)
    top_p = top_p / jnp.sum(top_p, axis=-1, keepdims=True)

    pos, te, tv = _meta(top_i.astype(jnp.int32))
    posf = pos.reshape(A_TOT)

    token_ids = jnp.arange(A_TOT, dtype=jnp.int32) % T
    token_of_pos = jnp.zeros((P_PAD,), jnp.int32).at[posf].set(token_ids)

    xs = xf[token_of_pos]
    eo = _grouped_ffn(xs, w1, b1, w2, b2, te, tv)

    out = (eo[posf[:T]] * top_p[:, 0:1] + eo[posf[T:]] * top_p[:, 1:2])
    return out.reshape(B, S, H)


# speedup vs baseline: 1.7608x; 1.3969x over previous
import functools

import jax
import jax.numpy as jnp
from jax import lax
from jax.experimental import pallas as pl
from jax.experimental.pallas import tpu as pltpu
from jax.experimental.pallas import tpu_sc as plsc

HIDDEN = 768
NUM_EXPERTS = 8
TOP_K = 2
D_FF = HIDDEN * 2
BM = 256
CH = 512
T_TOK = 2048
A_TOT = T_TOK * TOP_K
P_PAD = A_TOT + NUM_EXPERTS * BM
NTILES = P_PAD // BM

NC = 2
NS = 16
NW = NC * NS
APW = A_TOT // NW
TPW = T_TOK // NW
LANES = 16


def _meta_kernel(ti_ref, pos_ref, te_ref, tv_ref, rank_ref):
    T, E, e_i32 = T_TOK, NUM_EXPERTS, jnp.int32

    iota = lax.broadcasted_iota(e_i32, (T, E), 1)
    e1 = ti_ref[:, 0:1]
    e2 = ti_ref[:, 1:2]
    oh1 = (iota == e1).astype(jnp.float32)
    oh2 = (iota == e2).astype(jnp.float32)

    tri = (lax.broadcasted_iota(e_i32, (CH, CH), 0)
           > lax.broadcasted_iota(e_i32, (CH, CH), 1)).astype(jnp.bfloat16)
    base = jnp.zeros((1, E), jnp.float32)
    for c in range(A_TOT // CH):
        src = oh1 if c < T // CH else oh2
        ohf = src[(c % (T // CH)) * CH:((c % (T // CH)) + 1) * CH, :]
        partial = jnp.dot(tri, ohf.astype(jnp.bfloat16),
                          preferred_element_type=jnp.float32)
        rank_c = partial + base
        rank_ref[c * CH:(c + 1) * CH, :] = (
            jnp.sum(rank_c * ohf, axis=1, keepdims=True))
        base = base + jnp.sum(ohf, axis=0, keepdims=True)

    off = jnp.int32(0)
    starts, ends, offs = [], [], []
    for e in range(E):
        c_e = base[0, e].astype(e_i32)
        pc = ((c_e + BM - 1) // BM) * BM
        offs.append(off)
        starts.append(off // BM)
        ends.append((off + pc) // BM)
        off = off + pc
    for t in range(NTILES):
        te_t = jnp.int32(0)
        tv_t = jnp.int32(0)
        for e in range(E):
            inside = ((t >= starts[e]) & (t < ends[e])).astype(e_i32)
            te_t = te_t + inside * e
            tv_t = tv_t | inside
        te_ref[t] = te_t
        tv_ref[t] = tv_t

    iota8 = lax.broadcasted_iota(e_i32, (1, E), 1)
    offv = jnp.zeros((1, E), jnp.float32)
    for e in range(E):
        offv = jnp.where(iota8 == e, offs[e].astype(jnp.float32), offv)
    off1 = jnp.sum(oh1 * offv, axis=1, keepdims=True)
    off2 = jnp.sum(oh2 * offv, axis=1, keepdims=True)
    pos_ref[0:T, :] = (rank_ref[0:T, :] + off1).astype(e_i32)
    pos_ref[T:2 * T, :] = (rank_ref[T:2 * T, :] + off2).astype(e_i32)


def _meta(top_i):
    return pl.pallas_call(
        _meta_kernel,
        out_shape=[
            jax.ShapeDtypeStruct((A_TOT, 1), jnp.int32),
            jax.ShapeDtypeStruct((NTILES,), jnp.int32),
            jax.ShapeDtypeStruct((NTILES,), jnp.int32),
        ],
        out_specs=[
            pl.BlockSpec(memory_space=pltpu.VMEM),
            pl.BlockSpec(memory_space=pltpu.SMEM),
            pl.BlockSpec(memory_space=pltpu.SMEM),
        ],
        scratch_shapes=[pltpu.VMEM((A_TOT, 1), jnp.float32)],
    )(top_i)


def _dispatch_body(xf_hbm, pos_hbm, xs_hbm, idx_v, rows_v, sem):
    wid = lax.axis_index("s") * NC + lax.axis_index("c")
    base = wid * APW
    srow = lax.rem(base, T_TOK)
    pltpu.sync_copy(pos_hbm.at[pl.ds(base, APW)], idx_v)
    pltpu.sync_copy(xf_hbm.at[pl.ds(srow, APW), :], rows_v)
    pltpu.async_copy(rows_v, xs_hbm.at[idx_v], sem).wait()


def _combine_body(eo_hbm, pos_hbm, wexp_hbm, out_hbm, idx0_v, idx1_v, buf0,
                  buf1, w0_v, w1_v, sem0, sem1):
    wid = lax.axis_index("s") * NC + lax.axis_index("c")
    tb = wid * TPW
    pltpu.sync_copy(pos_hbm.at[pl.ds(tb, TPW)], idx0_v)
    pltpu.sync_copy(pos_hbm.at[pl.ds(T_TOK + tb, TPW)], idx1_v)
    cp0 = pltpu.async_copy(eo_hbm.at[idx0_v], buf0, sem0)
    cp1 = pltpu.async_copy(eo_hbm.at[idx1_v], buf1, sem1)
    pltpu.sync_copy(wexp_hbm.at[pl.ds(tb, TPW), :], w0_v)
    pltpu.sync_copy(wexp_hbm.at[pl.ds(T_TOK + tb, TPW), :], w1_v)
    cp0.wait()
    cp1.wait()

    def body(j, carry):
        w0 = w0_v[j, :]
        w1 = w1_v[j, :]
        for v in range(HIDDEN // LANES):
            sl = pl.ds(v * LANES, LANES)
            buf0[j, sl] = buf0[j, sl] * w0 + buf1[j, sl] * w1
        return carry

    lax.fori_loop(0, TPW, body, 0)
    pltpu.sync_copy(buf0, out_hbm.at[pl.ds(tb, TPW), :])


def _ffn_kernel(te_ref, tv_ref, xs_ref, w1_ref, b1_ref, w2_ref, b2_ref,
                out_ref):
    i = pl.program_id(0)

    @pl.when(tv_ref[i] > 0)
    def _():
        x_t = xs_ref[...].astype(jnp.bfloat16)
        h = jnp.dot(x_t, w1_ref[0].astype(jnp.bfloat16),
                    preferred_element_type=jnp.float32)
        h = jnp.maximum(h + b1_ref[0, 0, :][None, :], 0.0)
        o = jnp.dot(h.astype(jnp.bfloat16), w2_ref[0].astype(jnp.bfloat16),
                    preferred_element_type=jnp.float32)
        out_ref[...] = o + b2_ref[0, 0, :][None, :]


def _grouped_ffn(xs, w1, b1, w2, b2, tile_expert, tile_valid):
    grid_spec = pltpu.PrefetchScalarGridSpec(
        num_scalar_prefetch=2,
        grid=(NTILES,),
        in_specs=[
            pl.BlockSpec((BM, HIDDEN), lambda i, te, tv: (i, 0)),
            pl.BlockSpec((1, HIDDEN, D_FF), lambda i, te, tv: (te[i], 0, 0)),
            pl.BlockSpec((1, 1, D_FF), lambda i, te, tv: (te[i], 0, 0)),
            pl.BlockSpec((1, D_FF, HIDDEN), lambda i, te, tv: (te[i], 0, 0)),
            pl.BlockSpec((1, 1, HIDDEN), lambda i, te, tv: (te[i], 0, 0)),
        ],
        out_specs=pl.BlockSpec((BM, HIDDEN), lambda i, te, tv: (i, 0)),
    )
    return pl.pallas_call(
        _ffn_kernel,
        grid_spec=grid_spec,
        out_shape=jax.ShapeDtypeStruct((P_PAD, HIDDEN), jnp.float32),
    )(tile_expert, tile_valid, xs, w1, b1[:, None, :], w2, b2[:, None, :])


@jax.jit
def kernel(x, router_w, router_b, w1, b1, w2, b2):
    B, S, H = x.shape
    T = B * S
    xf = x.reshape(T, H)

    logits = xf @ router_w + router_b
    probs = jax.nn.softmax(logits, axis=-1)
    top_p, top_i = lax.top_k(probs, TOP_K)
    top_p = top_p / jnp.sum(top_p, axis=-1, keepdims=True)

    pos, te, tv = _meta(top_i.astype(jnp.int32))
    posf = pos.reshape(A_TOT)

    wexp = jnp.broadcast_to(top_p.T.reshape(A_TOT, 1), (A_TOT, LANES))

    mesh = plsc.VectorSubcoreMesh(core_axis_name="c", subcore_axis_name="s")
    dispatch = pl.kernel(
        _dispatch_body,
        mesh=mesh,
        out_type=jax.ShapeDtypeStruct((P_PAD, HIDDEN), jnp.float32),
        scratch_types=[
            pltpu.VMEM((APW,), jnp.int32),
            pltpu.VMEM((APW, HIDDEN), jnp.float32),
            pltpu.SemaphoreType.DMA,
        ],
    )
    combine = pl.kernel(
        _combine_body,
        mesh=mesh,
        out_type=jax.ShapeDtypeStruct((T_TOK, HIDDEN), jnp.float32),
        scratch_types=[
            pltpu.VMEM((TPW,), jnp.int32),
            pltpu.VMEM((TPW,), jnp.int32),
            pltpu.VMEM((TPW, HIDDEN), jnp.float32),
            pltpu.VMEM((TPW, HIDDEN), jnp.float32),
            pltpu.VMEM((TPW, LANES), jnp.float32),
            pltpu.VMEM((TPW, LANES), jnp.float32),
            pltpu.SemaphoreType.DMA,
            pltpu.SemaphoreType.DMA,
        ],
    )

    xs = dispatch(xf, posf)
    eo = _grouped_ffn(xs, w1, b1, w2, b2, te, tv)
    out = combine(eo, posf, wexp)
    return out.reshape(B, S, H)
